# Initial kernel scaffold; baseline (speedup 1.0000x reference)
import jax
import jax.numpy as jnp
from jax.experimental import pallas as pl


def kernel(x, W_src, W_dst, attn_l, attn_r, bias, edge_index, edge_type):
    raise NotImplementedError("write your pallas kernel here")


# speedup vs baseline: 114.6597x; 114.6597x over previous
import functools

import jax
import jax.numpy as jnp
from jax import lax
from jax.experimental import pallas as pl
from jax.experimental.pallas import tpu as pltpu
from jax.experimental.pallas import tpu_sc as plsc

N = 10000
E = 320000
D = 128
R = 4
L = 3
NP = 10240
NPAD = 10240
RNT = R * NP
NC = 2
NS = 16
NW = NC * NS
EPW = E // NW
G = 80
NG = EPW // G
CH = 2000
NEG = -3.0e38

_mesh = plsc.VectorSubcoreMesh(core_axis_name="c", subcore_axis_name="s")


def _rel_max(tab):
    def body(j, acc):
        return jnp.maximum(acc, tab[pl.ds(j * 16, 16)])
    return lax.fori_loop(0, N * R // 16, body,
                         jnp.full((16,), NEG, jnp.float32))


def _per_rel(acc):
    lanes = lax.iota(jnp.int32, 16)
    out = jnp.full((16,), 0.0, jnp.float32)
    for r in range(R):
        m_r = jnp.maximum(jnp.maximum(acc[r], acc[r + R]),
                          jnp.maximum(acc[r + 2 * R], acc[r + 3 * R]))
        out = jnp.where(lanes == r, m_r, out)
    return out


@functools.partial(
    pl.kernel,
    out_type=[
        jax.ShapeDtypeStruct((E,), jnp.float32),
        jax.ShapeDtypeStruct((NC * RNT,), jnp.float32),
    ],
    mesh=_mesh,
    compiler_params=pltpu.CompilerParams(needs_layout_passes=False),
    scratch_types=[
        pltpu.VMEM((R * N,), jnp.float32),
        pltpu.VMEM((EPW,), jnp.float32),
        pltpu.VMEM((EPW,), jnp.int32),
        pltpu.VMEM((EPW,), jnp.int32),
        pltpu.VMEM((G,), jnp.int32),
        pltpu.VMEM((G,), jnp.int32),
        pltpu.VMEM((G,), jnp.float32),
        pltpu.VMEM((G,), jnp.float32),
        pltpu.VMEM((16,), jnp.float32),
        pltpu.VMEM((RNT // NS,), jnp.float32),
        pltpu.VMEM_SHARED((RNT,), jnp.float32),
        pltpu.SemaphoreType.DMA,
        pltpu.SemaphoreType.DMA,
    ],
)
def _sc_pass1(el_hbm, er_hbm, src_hbm, dst_hbm, et_hbm, ex_hbm, den_hbm,
              tab, sbuf, ibufA, ibufB, seg0, seg1, exr0, exr1, mbuf, zbuf,
              den_sh, sem0, sem1):
    c = lax.axis_index("c")
    s = lax.axis_index("s")
    wid = c * NS + s
    base = wid * EPW
    segs = (seg0, seg1)
    exrs = (exr0, exr1)
    sems = (sem0, sem1)

    pltpu.sync_copy(el_hbm, tab)
    pltpu.sync_copy(src_hbm.at[pl.ds(base, EPW)], ibufA)
    pltpu.sync_copy(et_hbm.at[pl.ds(base, EPW)], ibufB)

    mel = _per_rel(_rel_max(tab))

    def bodyA(j, _):
        sv = ibufA[pl.ds(j * 16, 16)]
        ev = ibufB[pl.ds(j * 16, 16)]
        g = plsc.load_gather(tab, [sv * R + ev])
        sbuf[pl.ds(j * 16, 16)] = g
        return 0
    lax.fori_loop(0, EPW // 16, bodyA, 0)

    pltpu.sync_copy(er_hbm, tab)
    pltpu.sync_copy(dst_hbm.at[pl.ds(base, EPW)], ibufA)

    mer = _per_rel(_rel_max(tab))
    msum = mel + mer
    mbuf[...] = jnp.maximum(msum, 0.2 * msum)

    zc = RNT // NS
    def zbody(j, _):
        zbuf[pl.ds(j * 16, 16)] = jnp.zeros((16,), jnp.float32)
        return 0
    lax.fori_loop(0, zc // 16, zbody, 0)
    pltpu.sync_copy(zbuf, den_sh.at[pl.ds(s * zc, zc)])
    plsc.subcore_barrier()

    def group(g_idx, b):
        def body(j, _):
            i = g_idx * G + j * 16
            dv = ibufA[pl.ds(i, 16)]
            ev = ibufB[pl.ds(i, 16)]
            erg = plsc.load_gather(tab, [dv * R + ev])
            sval = sbuf[pl.ds(i, 16)] + erg
            sc = jnp.maximum(sval, 0.2 * sval)
            mg = plsc.load_gather(mbuf, [ev])
            exv = jnp.exp(sc - mg)
            sbuf[pl.ds(i, 16)] = exv
            segs[b][pl.ds(j * 16, 16)] = ev * NP + dv
            exrs[b][pl.ds(j * 16, 16)] = exv
            return 0
        lax.fori_loop(0, G // 16, body, 0)
        pltpu.async_copy(exrs[b], den_sh.at[segs[b]], sems[b], add=True)

    def wait(b):
        pltpu.make_async_copy(exrs[b], den_sh.at[segs[b]], sems[b]).wait()

    group(0, 0)
    group(1, 1)

    def pair(jj, _):
        for b in range(2):
            wait(b)
            group(2 * jj + b, b)
        return 0
    lax.fori_loop(1, (NG - 1) // 2, pair, 0)
    wait(0)
    group(NG - 1, 0)
    wait(0)
    wait(1)

    pltpu.sync_copy(sbuf, ex_hbm.at[pl.ds(base, EPW)])
    plsc.subcore_barrier()
    pltpu.sync_copy(den_sh.at[pl.ds(s * zc, zc)],
                    den_hbm.at[pl.ds(c * RNT + s * zc, zc)])


@functools.partial(
    pl.kernel,
    out_type=jax.ShapeDtypeStruct((E,), jnp.float32),
    mesh=_mesh,
    compiler_params=pltpu.CompilerParams(needs_layout_passes=False),
    scratch_types=[
        pltpu.VMEM((RNT,), jnp.float32),
        pltpu.VMEM((2048,), jnp.float32),
        pltpu.VMEM((CH,), jnp.float32),
        pltpu.VMEM((CH,), jnp.int32),
        pltpu.VMEM((CH,), jnp.int32),
        pltpu.VMEM((CH,), jnp.float32),
    ],
)
def _sc_pass1b(ex_hbm, den_hbm, dst_hbm, et_hbm, al_hbm,
               den, dtmp, exc, dstc, etc_, alc):
    c = lax.axis_index("c")
    s = lax.axis_index("s")
    wid = c * NS + s
    base = wid * EPW

    pltpu.sync_copy(den_hbm.at[pl.ds(0, RNT)], den)
    for k in range(RNT // 2048):
        pltpu.sync_copy(den_hbm.at[pl.ds(RNT + k * 2048, 2048)], dtmp)
        def addb(j, _):
            i = k * 2048 + j * 16
            den[pl.ds(i, 16)] = den[pl.ds(i, 16)] + dtmp[pl.ds(j * 16, 16)]
            return 0
        lax.fori_loop(0, 128, addb, 0)

    for kc in range(EPW // CH):
        off = base + kc * CH
        pltpu.sync_copy(ex_hbm.at[pl.ds(off, CH)], exc)
        pltpu.sync_copy(dst_hbm.at[pl.ds(off, CH)], dstc)
        pltpu.sync_copy(et_hbm.at[pl.ds(off, CH)], etc_)
        def body(j, _):
            sl = pl.ds(j * 16, 16)
            dv = dstc[sl]
            ev = etc_[sl]
            exv = exc[sl]
            dg = plsc.load_gather(den, [ev * NP + dv])
            alc[sl] = exv / jnp.maximum(dg, 1e-30)
            return 0
        lax.fori_loop(0, CH // 16, body, 0)
        pltpu.sync_copy(alc, al_hbm.at[pl.ds(off, CH)])


@functools.partial(
    pl.kernel,
    out_type=jax.ShapeDtypeStruct((NC * NPAD, D), jnp.float32),
    mesh=_mesh,
    compiler_params=pltpu.CompilerParams(needs_layout_passes=False),
    scratch_types=[
        pltpu.VMEM((CH,), jnp.float32),
        pltpu.VMEM((CH,), jnp.int32),
        pltpu.VMEM((CH,), jnp.int32),
        pltpu.VMEM((CH,), jnp.int32),
        pltpu.VMEM((G, D), jnp.float32),
        pltpu.VMEM((G, D), jnp.float32),
        pltpu.VMEM((G,), jnp.int32),
        pltpu.VMEM((G,), jnp.int32),
        pltpu.VMEM((G,), jnp.int32),
        pltpu.VMEM((G,), jnp.int32),
        pltpu.VMEM((G,), jnp.float32),
        pltpu.VMEM((G,), jnp.float32),
        pltpu.VMEM_SHARED((NPAD, D), jnp.float32),
        pltpu.SemaphoreType.DMA,
        pltpu.SemaphoreType.DMA,
        pltpu.SemaphoreType.DMA,
        pltpu.SemaphoreType.DMA,
    ],
)
def _sc_pass2(al_hbm, fs_hbm, src_hbm, dst_hbm, et_hbm, out_hbm,
              alf, sbufA, sbufC, sbufD, rows0, rows1,
              gi0, gi1, si0, si1, al0, al1, out_sh,
              gsem0, gsem1, ssem0, ssem1):
    c = lax.axis_index("c")
    s = lax.axis_index("s")
    wid = c * NS + s
    base = wid * EPW
    rows = (rows0, rows1)
    gis = (gi0, gi1)
    sis = (si0, si1)
    als = (al0, al1)
    gsems = (gsem0, gsem1)
    ssems = (ssem0, ssem1)

    def zrow(q, _):
        i = q // (D // 16)
        cc = q % (D // 16)
        rows0[i, pl.ds(cc * 16, 16)] = jnp.zeros((16,), jnp.float32)
        return 0
    lax.fori_loop(0, G * (D // 16), zrow, 0)
    npw = NPAD // NS
    for t in range(npw // G):
        pltpu.sync_copy(rows0, out_sh.at[pl.ds(s * npw + t * G, G)])
    plsc.subcore_barrier()

    GPC = CH // G

    def prep(g_idx, b):
        def body(j, _):
            i = g_idx * G + j * 16
            sv = sbufA[pl.ds(i, 16)]
            ev = sbufC[pl.ds(i, 16)]
            dv = sbufD[pl.ds(i, 16)]
            gis[b][pl.ds(j * 16, 16)] = ev * N + sv
            sis[b][pl.ds(j * 16, 16)] = dv
            als[b][pl.ds(j * 16, 16)] = alf[pl.ds(i, 16)]
            return 0
        lax.fori_loop(0, G // 16, body, 0)
        pltpu.async_copy(fs_hbm.at[gis[b]], rows[b], gsems[b])

    def scale_and_scatter(b):
        pltpu.make_async_copy(fs_hbm.at[gis[b]], rows[b], gsems[b]).wait()
        def sbody(q, _):
            av = als[b][pl.ds(q * 16, 16)]
            for k in range(16):
                a = av[k]
                i = q * 16 + k
                for cc in range(D // 16):
                    sl = pl.ds(cc * 16, 16)
                    rows[b][i, sl] = rows[b][i, sl] * a
            return 0
        lax.fori_loop(0, G // 16, sbody, 0)
        pltpu.async_copy(rows[b], out_sh.at[sis[b]], ssems[b], add=True)

    def swait(b):
        pltpu.make_async_copy(rows[b], out_sh.at[sis[b]], ssems[b]).wait()

    for kc in range(EPW // CH):
        off = base + kc * CH
        pltpu.sync_copy(al_hbm.at[pl.ds(off, CH)], alf)
        pltpu.sync_copy(src_hbm.at[pl.ds(off, CH)], sbufA)
        pltpu.sync_copy(et_hbm.at[pl.ds(off, CH)], sbufC)
        pltpu.sync_copy(dst_hbm.at[pl.ds(off, CH)], sbufD)

        prep(0, 0)
        prep(1, 1)

        def pair(jj, _):
            for b in range(2):
                g_idx = 2 * jj + b
                scale_and_scatter(b)
                swait(b)

                @pl.when(g_idx + 2 < GPC)
                def _():
                    prep(g_idx + 2, b)
            return 0
        lax.fori_loop(0, (GPC - 1) // 2, pair, 0)
        scale_and_scatter(0)
        swait(0)

    plsc.subcore_barrier()
    pltpu.sync_copy(out_sh.at[pl.ds(s * npw, npw)],
                    out_hbm.at[pl.ds(c * NPAD + s * npw, npw)])


_NB = 2000


def _tc_prelayer_body(h_ref, ws_ref, wd_ref, al_ref, ar_ref,
                      fs_ref, el_ref, er_ref):
    h = h_ref[...]
    al = al_ref[...]
    ar = ar_ref[...]
    els = []
    ers = []
    for r in range(R):
        fsr = jnp.dot(h, ws_ref[r], preferred_element_type=jnp.float32)
        fs_ref[r] = fsr
        els.append(jnp.sum(fsr * al[r][None, :], axis=1))
        hw = jnp.dot(h, wd_ref[r], preferred_element_type=jnp.float32)
        ers.append(jnp.sum(hw * ar[r][None, :], axis=1))
    el_ref[...] = jnp.stack(els, axis=1)
    er_ref[...] = jnp.stack(ers, axis=1)


def _tc_prelayer(h, ws, wd, al, ar):
    return pl.pallas_call(
        _tc_prelayer_body,
        grid=(N // _NB,),
        in_specs=[
            pl.BlockSpec((_NB, D), lambda i: (i, 0)),
            pl.BlockSpec((R, D, D), lambda i: (0, 0, 0)),
            pl.BlockSpec((R, D, D), lambda i: (0, 0, 0)),
            pl.BlockSpec((R, D), lambda i: (0, 0)),
            pl.BlockSpec((R, D), lambda i: (0, 0)),
        ],
        out_specs=[
            pl.BlockSpec((R, _NB, D), lambda i: (0, i, 0)),
            pl.BlockSpec((_NB, R), lambda i: (i, 0)),
            pl.BlockSpec((_NB, R), lambda i: (i, 0)),
        ],
        out_shape=[
            jax.ShapeDtypeStruct((R, N, D), jnp.float32),
            jax.ShapeDtypeStruct((N, R), jnp.float32),
            jax.ShapeDtypeStruct((N, R), jnp.float32),
        ],
    )(h, ws, wd, al, ar)


def _tc_combine_body(relu, p0_ref, p1_ref, b_ref, h_ref):
    v = p0_ref[...] + p1_ref[...] + b_ref[...]
    if relu:
        v = jnp.maximum(v, 0.0)
    h_ref[...] = v


def _tc_combine(p0, p1, b, relu):
    return pl.pallas_call(
        functools.partial(_tc_combine_body, relu),
        grid=(N // _NB,),
        in_specs=[
            pl.BlockSpec((_NB, D), lambda i: (i, 0)),
            pl.BlockSpec((_NB, D), lambda i: (i, 0)),
            pl.BlockSpec((1, D), lambda i: (0, 0)),
        ],
        out_specs=pl.BlockSpec((_NB, D), lambda i: (i, 0)),
        out_shape=jax.ShapeDtypeStruct((N, D), jnp.float32),
    )(p0, p1, b)


def kernel(x, W_src, W_dst, attn_l, attn_r, bias, edge_index, edge_type):
    src = edge_index[0].astype(jnp.int32)
    dst = edge_index[1].astype(jnp.int32)
    et = edge_type.astype(jnp.int32)
    h = x
    for l in range(L):
        fs, el, er = _tc_prelayer(h, W_src[l], W_dst[l], attn_l[l], attn_r[l])
        ex, den = _sc_pass1(el.reshape(N * R), er.reshape(N * R),
                            src, dst, et)
        alpha = _sc_pass1b(ex, den, dst, et)
        p = _sc_pass2(alpha, fs.reshape(R * N, D), src, dst, et)
        h = _tc_combine(p[:N], p[NPAD:NPAD + N], bias[l][None, :],
                        relu=(l < L - 1))
    return h


# speedup vs baseline: 119.9249x; 1.0459x over previous
import functools

import jax
import jax.numpy as jnp
from jax import lax
from jax.experimental import pallas as pl
from jax.experimental.pallas import tpu as pltpu
from jax.experimental.pallas import tpu_sc as plsc

N = 10000
E = 320000
D = 128
R = 4
L = 3
NP = 10240
NPAD = 10240
RNT = R * NP
NC = 2
NS = 16
NW = NC * NS
EPW = E // NW
G = 80
NG = EPW // G
CH = 2000
NEG = -3.0e38

_mesh = plsc.VectorSubcoreMesh(core_axis_name="c", subcore_axis_name="s")


@functools.partial(
    pl.kernel,
    out_type=[
        jax.ShapeDtypeStruct((E,), jnp.float32),
        jax.ShapeDtypeStruct((NC * RNT,), jnp.float32),
    ],
    mesh=_mesh,
    compiler_params=pltpu.CompilerParams(needs_layout_passes=False),
    scratch_types=[
        pltpu.VMEM((R * N,), jnp.float32),
        pltpu.VMEM((EPW,), jnp.float32),
        pltpu.VMEM((EPW,), jnp.int32),
        pltpu.VMEM((EPW,), jnp.int32),
        pltpu.VMEM((G,), jnp.int32),
        pltpu.VMEM((G,), jnp.int32),
        pltpu.VMEM((G,), jnp.float32),
        pltpu.VMEM((G,), jnp.float32),
        pltpu.VMEM((16,), jnp.float32),
        pltpu.VMEM((RNT // NS,), jnp.float32),
        pltpu.VMEM_SHARED((RNT,), jnp.float32),
        pltpu.SemaphoreType.DMA,
        pltpu.SemaphoreType.DMA,
    ],
)
def _sc_pass1(el_hbm, er_hbm, mx_hbm, src_hbm, dst_hbm, et_hbm,
              ex_hbm, den_hbm,
              tab, sbuf, ibufA, ibufB, seg0, seg1, exr0, exr1, mbuf, zbuf,
              den_sh, sem0, sem1):
    c = lax.axis_index("c")
    s = lax.axis_index("s")
    wid = c * NS + s
    base = wid * EPW
    segs = (seg0, seg1)
    exrs = (exr0, exr1)
    sems = (sem0, sem1)

    pltpu.sync_copy(el_hbm, tab)
    pltpu.sync_copy(src_hbm.at[pl.ds(base, EPW)], ibufA)
    pltpu.sync_copy(et_hbm.at[pl.ds(base, EPW)], ibufB)
    pltpu.sync_copy(mx_hbm, mbuf)

    def bodyA(j, _):
        sv = ibufA[pl.ds(j * 16, 16)]
        ev = ibufB[pl.ds(j * 16, 16)]
        g = plsc.load_gather(tab, [sv * R + ev])
        sbuf[pl.ds(j * 16, 16)] = g
        return 0
    lax.fori_loop(0, EPW // 16, bodyA, 0)

    pltpu.sync_copy(er_hbm, tab)
    pltpu.sync_copy(dst_hbm.at[pl.ds(base, EPW)], ibufA)

    lanes = lax.iota(jnp.int32, 16)
    mv = mbuf[...]
    msum = jnp.full((16,), 0.0, jnp.float32)
    for r in range(R):
        msum = jnp.where(lanes == r, mv[r] + mv[R + r], msum)
    mbuf[...] = jnp.maximum(msum, 0.2 * msum)

    zc = RNT // NS
    def zbody(j, _):
        zbuf[pl.ds(j * 16, 16)] = jnp.zeros((16,), jnp.float32)
        return 0
    lax.fori_loop(0, zc // 16, zbody, 0)
    pltpu.sync_copy(zbuf, den_sh.at[pl.ds(s * zc, zc)])
    plsc.subcore_barrier()

    def group(g_idx, b):
        def body(j, _):
            i = g_idx * G + j * 16
            dv = ibufA[pl.ds(i, 16)]
            ev = ibufB[pl.ds(i, 16)]
            erg = plsc.load_gather(tab, [dv * R + ev])
            sval = sbuf[pl.ds(i, 16)] + erg
            sc = jnp.maximum(sval, 0.2 * sval)
            mg = plsc.load_gather(mbuf, [ev])
            exv = jnp.exp(sc - mg)
            sbuf[pl.ds(i, 16)] = exv
            segs[b][pl.ds(j * 16, 16)] = ev * NP + dv
            exrs[b][pl.ds(j * 16, 16)] = exv
            return 0
        lax.fori_loop(0, G // 16, body, 0)
        pltpu.async_copy(exrs[b], den_sh.at[segs[b]], sems[b], add=True)

    def wait(b):
        pltpu.make_async_copy(exrs[b], den_sh.at[segs[b]], sems[b]).wait()

    group(0, 0)
    group(1, 1)

    def pair(jj, _):
        for b in range(2):
            wait(b)
            group(2 * jj + b, b)
        return 0
    lax.fori_loop(1, (NG - 1) // 2, pair, 0)
    wait(0)
    group(NG - 1, 0)
    wait(0)
    wait(1)

    pltpu.sync_copy(sbuf, ex_hbm.at[pl.ds(base, EPW)])
    plsc.subcore_barrier()
    pltpu.sync_copy(den_sh.at[pl.ds(s * zc, zc)],
                    den_hbm.at[pl.ds(c * RNT + s * zc, zc)])


@functools.partial(
    pl.kernel,
    out_type=[
        jax.ShapeDtypeStruct((E,), jnp.float32),
        jax.ShapeDtypeStruct((E,), jnp.int32),
    ],
    mesh=_mesh,
    compiler_params=pltpu.CompilerParams(needs_layout_passes=False),
    scratch_types=[
        pltpu.VMEM((RNT,), jnp.float32),
        pltpu.VMEM((2048,), jnp.float32),
        pltpu.VMEM((CH,), jnp.float32),
        pltpu.VMEM((CH,), jnp.int32),
        pltpu.VMEM((CH,), jnp.int32),
        pltpu.VMEM((CH,), jnp.int32),
        pltpu.VMEM((CH,), jnp.float32),
        pltpu.VMEM((CH,), jnp.int32),
    ],
)
def _sc_pass1b(ex_hbm, den_hbm, src_hbm, dst_hbm, et_hbm, al_hbm, gi_hbm,
               den, dtmp, exc, srcc, dstc, etc_, alc, gic):
    c = lax.axis_index("c")
    s = lax.axis_index("s")
    wid = c * NS + s
    base = wid * EPW

    pltpu.sync_copy(den_hbm.at[pl.ds(0, RNT)], den)
    for k in range(RNT // 2048):
        pltpu.sync_copy(den_hbm.at[pl.ds(RNT + k * 2048, 2048)], dtmp)
        def addb(j, _):
            i = k * 2048 + j * 16
            den[pl.ds(i, 16)] = den[pl.ds(i, 16)] + dtmp[pl.ds(j * 16, 16)]
            return 0
        lax.fori_loop(0, 128, addb, 0)

    for kc in range(EPW // CH):
        off = base + kc * CH
        pltpu.sync_copy(ex_hbm.at[pl.ds(off, CH)], exc)
        pltpu.sync_copy(src_hbm.at[pl.ds(off, CH)], srcc)
        pltpu.sync_copy(dst_hbm.at[pl.ds(off, CH)], dstc)
        pltpu.sync_copy(et_hbm.at[pl.ds(off, CH)], etc_)
        def body(j, _):
            sl = pl.ds(j * 16, 16)
            dv = dstc[sl]
            ev = etc_[sl]
            exv = exc[sl]
            dg = plsc.load_gather(den, [ev * NP + dv])
            alc[sl] = exv / jnp.maximum(dg, 1e-30)
            gic[sl] = ev * N + srcc[sl]
            return 0
        lax.fori_loop(0, CH // 16, body, 0)
        pltpu.sync_copy(alc, al_hbm.at[pl.ds(off, CH)])
        pltpu.sync_copy(gic, gi_hbm.at[pl.ds(off, CH)])


@functools.partial(
    pl.kernel,
    out_type=jax.ShapeDtypeStruct((NC * NPAD, D), jnp.float32),
    mesh=_mesh,
    compiler_params=pltpu.CompilerParams(needs_layout_passes=False),
    scratch_types=[
        pltpu.VMEM((CH,), jnp.float32),
        pltpu.VMEM((CH,), jnp.int32),
        pltpu.VMEM((CH,), jnp.int32),
        pltpu.VMEM((CH // G, G), jnp.int32),
        pltpu.VMEM((G, D), jnp.float32),
        pltpu.VMEM((G, D), jnp.float32),
        pltpu.VMEM((G, D), jnp.float32),
        pltpu.VMEM_SHARED((NPAD, D), jnp.float32),
        pltpu.SemaphoreType.DMA,
        pltpu.SemaphoreType.DMA,
        pltpu.SemaphoreType.DMA,
        pltpu.SemaphoreType.DMA,
        pltpu.SemaphoreType.DMA,
        pltpu.SemaphoreType.DMA,
    ],
)
def _sc_pass2(al_hbm, gi_hbm, dst_hbm, fs_hbm, out_hbm,
              alf, gif, alf2i, si2, rows0, rows1, rows2, out_sh,
              gsem0, gsem1, gsem2, ssem0, ssem1, ssem2):
    c = lax.axis_index("c")
    s = lax.axis_index("s")
    wid = c * NS + s
    base = wid * EPW
    rows = (rows0, rows1, rows2)
    gsems = (gsem0, gsem1, gsem2)
    ssems = (ssem0, ssem1, ssem2)
    ND = 3

    def zrow(q, _):
        i = q // (D // 16)
        cc = q % (D // 16)
        rows0[i, pl.ds(cc * 16, 16)] = jnp.zeros((16,), jnp.float32)
        return 0
    lax.fori_loop(0, G * (D // 16), zrow, 0)
    npw = NPAD // NS
    for t in range(npw // G):
        pltpu.sync_copy(rows0, out_sh.at[pl.ds(s * npw + t * G, G)])
    plsc.subcore_barrier()

    GPC = CH // G

    def fire_gather(g_idx, b):
        pltpu.async_copy(fs_hbm.at[gif.at[pl.ds(g_idx * G, G)]],
                         rows[b], gsems[b])

    def gwait(b):
        pltpu.make_async_copy(fs_hbm.at[gif.at[pl.ds(0, G)]],
                              rows[b], gsems[b]).wait()

    def scale_and_scatter(g_idx, b):
        gwait(b)
        def sbody(q, _):
            av = alf[pl.ds(g_idx * G + q * 16, 16)]
            for k in range(16):
                a = av[k]
                i = q * 16 + k
                for cc in range(D // 16):
                    sl = pl.ds(cc * 16, 16)
                    rows[b][i, sl] = rows[b][i, sl] * a
            return 0
        lax.fori_loop(0, G // 16, sbody, 0)
        pltpu.async_copy(rows[b], out_sh.at[si2.at[g_idx]], ssems[b],
                         add=True)

    def swait(b):
        pltpu.make_async_copy(rows[b], out_sh.at[si2.at[0]], ssems[b]).wait()

    def chunk(kc, _):
        off = base + kc * CH
        pltpu.sync_copy(al_hbm.at[pl.ds(off, CH)], alf)
        pltpu.sync_copy(gi_hbm.at[pl.ds(off, CH)], gif)
        pltpu.sync_copy(dst_hbm.at[pl.ds(off, CH)], alf2i)

        def sicopy(q, _):
            row = q // (G // 16)
            col = q % (G // 16)
            si2[row, pl.ds(col * 16, 16)] = alf2i[pl.ds(q * 16, 16)]
            return 0
        lax.fori_loop(0, CH // 16, sicopy, 0)

        for b in range(ND):
            fire_gather(b, b)

        def ring(jj, _):
            for b in range(ND):
                scale_and_scatter(ND * jj + b - ND, b)
            for b in range(ND):
                g_idx = ND * jj + b
                swait(b)
                fire_gather(g_idx, b)
            return 0
        lax.fori_loop(1, GPC // ND, ring, 0)
        for b in range(ND):
            scale_and_scatter(GPC - ND - 1 + b, b)
        swait(0)
        fire_gather(GPC - 1, 0)
        scale_and_scatter(GPC - 1, 0)
        for b in range(ND):
            swait(b)
        return 0
    lax.fori_loop(0, EPW // CH, chunk, 0)

    plsc.subcore_barrier()
    pltpu.sync_copy(out_sh.at[pl.ds(s * npw, npw)],
                    out_hbm.at[pl.ds(c * NPAD + s * npw, npw)])


_NB = 2000


def _prelayer_compute(h, ws_ref, wd_ref, al, ar, fs_ref, el_ref, er_ref,
                      mx_ref, first):
    els = []
    ers = []
    for r in range(R):
        fsr = jnp.dot(h, ws_ref[r], preferred_element_type=jnp.float32)
        fs_ref[r] = fsr
        els.append(jnp.sum(fsr * al[r][None, :], axis=1))
        hw = jnp.dot(h, wd_ref[r], preferred_element_type=jnp.float32)
        ers.append(jnp.sum(hw * ar[r][None, :], axis=1))
    el = jnp.stack(els, axis=1)
    er = jnp.stack(ers, axis=1)
    el_ref[...] = el
    er_ref[...] = er
    lanes8 = lax.iota(jnp.int32, 16)
    cur = jnp.full((16,), NEG, jnp.float32)
    for r in range(R):
        cur = jnp.where(lanes8 == r, jnp.max(el[:, r]), cur)
        cur = jnp.where(lanes8 == R + r, jnp.max(er[:, r]), cur)

    @pl.when(first)
    def _():
        mx_ref[...] = cur

    @pl.when(jnp.logical_not(first))
    def _():
        mx_ref[...] = jnp.maximum(mx_ref[...], cur)


def _tc_prelayer_body(h_ref, ws_ref, wd_ref, al_ref, ar_ref,
                      fs_ref, el_ref, er_ref, mx_ref):
    first = pl.program_id(0) == 0
    _prelayer_compute(h_ref[...], ws_ref, wd_ref, al_ref[...], ar_ref[...],
                      fs_ref, el_ref, er_ref, mx_ref, first)


def _tc_prelayer2_body(p0_ref, p1_ref, b_ref, ws_ref, wd_ref, al_ref, ar_ref,
                       fs_ref, el_ref, er_ref, mx_ref):
    first = pl.program_id(0) == 0
    h = jnp.maximum(p0_ref[...] + p1_ref[...] + b_ref[...], 0.0)
    _prelayer_compute(h, ws_ref, wd_ref, al_ref[...], ar_ref[...],
                      fs_ref, el_ref, er_ref, mx_ref, first)


_PRE_OUT_SPECS = [
    pl.BlockSpec((R, _NB, D), lambda i: (0, i, 0)),
    pl.BlockSpec((_NB, R), lambda i: (i, 0)),
    pl.BlockSpec((_NB, R), lambda i: (i, 0)),
    pl.BlockSpec((16,), lambda i: (0,)),
]
_PRE_OUT_SHAPE = [
    jax.ShapeDtypeStruct((R, N, D), jnp.float32),
    jax.ShapeDtypeStruct((N, R), jnp.float32),
    jax.ShapeDtypeStruct((N, R), jnp.float32),
    jax.ShapeDtypeStruct((16,), jnp.float32),
]
_W_SPECS = [
    pl.BlockSpec((R, D, D), lambda i: (0, 0, 0)),
    pl.BlockSpec((R, D, D), lambda i: (0, 0, 0)),
    pl.BlockSpec((R, D), lambda i: (0, 0)),
    pl.BlockSpec((R, D), lambda i: (0, 0)),
]


def _tc_prelayer(h, ws, wd, al, ar):
    return pl.pallas_call(
        _tc_prelayer_body,
        grid=(N // _NB,),
        in_specs=[pl.BlockSpec((_NB, D), lambda i: (i, 0))] + _W_SPECS,
        out_specs=_PRE_OUT_SPECS,
        out_shape=_PRE_OUT_SHAPE,
    )(h, ws, wd, al, ar)


def _tc_prelayer2(p0, p1, b, ws, wd, al, ar):
    return pl.pallas_call(
        _tc_prelayer2_body,
        grid=(N // _NB,),
        in_specs=[
            pl.BlockSpec((_NB, D), lambda i: (i, 0)),
            pl.BlockSpec((_NB, D), lambda i: (i, 0)),
            pl.BlockSpec((1, D), lambda i: (0, 0)),
        ] + _W_SPECS,
        out_specs=_PRE_OUT_SPECS,
        out_shape=_PRE_OUT_SHAPE,
    )(p0, p1, b, ws, wd, al, ar)


def _tc_combine_body(relu, p0_ref, p1_ref, b_ref, h_ref):
    v = p0_ref[...] + p1_ref[...] + b_ref[...]
    if relu:
        v = jnp.maximum(v, 0.0)
    h_ref[...] = v


def _tc_combine(p0, p1, b, relu):
    return pl.pallas_call(
        functools.partial(_tc_combine_body, relu),
        grid=(N // _NB,),
        in_specs=[
            pl.BlockSpec((_NB, D), lambda i: (i, 0)),
            pl.BlockSpec((_NB, D), lambda i: (i, 0)),
            pl.BlockSpec((1, D), lambda i: (0, 0)),
        ],
        out_specs=pl.BlockSpec((_NB, D), lambda i: (i, 0)),
        out_shape=jax.ShapeDtypeStruct((N, D), jnp.float32),
    )(p0, p1, b)


def kernel(x, W_src, W_dst, attn_l, attn_r, bias, edge_index, edge_type):
    src = edge_index[0].astype(jnp.int32)
    dst = edge_index[1].astype(jnp.int32)
    et = edge_type.astype(jnp.int32)
    p = None
    for l in range(L):
        if l == 0:
            fs, el, er, mx = _tc_prelayer(x, W_src[l], W_dst[l],
                                          attn_l[l], attn_r[l])
        else:
            fs, el, er, mx = _tc_prelayer2(p[:N], p[NPAD:NPAD + N],
                                           bias[l - 1][None, :],
                                           W_src[l], W_dst[l],
                                           attn_l[l], attn_r[l])
        ex, den = _sc_pass1(el.reshape(N * R), er.reshape(N * R), mx,
                            src, dst, et)
        alpha, gidx = _sc_pass1b(ex, den, src, dst, et)
        p = _sc_pass2(alpha, gidx, dst, fs.reshape(R * N, D))
    return _tc_combine(p[:N], p[NPAD:NPAD + N], bias[L - 1][None, :],
                       relu=False)
